# baseline (speedup 1.0000x reference)
import functools

import jax
import jax.numpy as jnp
from jax import lax
from jax.experimental import pallas as pl
from jax.experimental.pallas import tpu as pltpu
from jax.experimental.pallas import tpu_sc as plsc

BATCH = 16384
DIM = 64
NUM_CORES = 1
NUM_SUBCORES = 16
NW = NUM_CORES * NUM_SUBCORES
B_PER_W = BATCH // NW
CHUNK = 128
NCHUNK = B_PER_W // CHUNK
GRP = 16


def _gather_body(lidx_hbm, par_hbm, tab_hbm, out_hbm,
                 lidx_v, par_v, pairs_a, pairs_b, rows_a, rows_b,
                 sem_a, sem_b, osem_a, osem_b):
    wid = lax.axis_index("s") * NUM_CORES + lax.axis_index("c")
    base = wid * B_PER_W
    pltpu.sync_copy(lidx_hbm.at[wid], lidx_v)
    pltpu.sync_copy(par_hbm.at[wid], par_v)
    pair_bufs = (pairs_a, pairs_b)
    pair_sems = (sem_a, sem_b)
    row_bufs = (rows_a, rows_b)
    out_sems = (osem_a, osem_b)

    def fire(j, slot):
        return pltpu.async_copy(
            tab_hbm.at[lidx_v.at[j]], pair_bufs[slot], pair_sems[slot]
        )

    def extract(j, slot):
        buf = pair_bufs[slot]
        rows = row_bufs[slot]

        def grp_body(g, carry):
            voff = par_v[pl.ds(j * CHUNK + g * GRP, GRP)] * DIM
            for l in range(GRP):
                off = voff[l]
                for c in range(DIM // 16):
                    rows[g * GRP + l, pl.ds(c * 16, 16)] = (
                        buf[g * GRP + l, pl.ds(off + c * 16, 16)]
                    )
            return carry

        lax.fori_loop(0, CHUNK // GRP, grp_body, 0)

    out_copies = [None, None]
    pending = fire(0, 0)
    for j in range(NCHUNK):
        slot = j % 2
        cur = pending
        pending = fire(j + 1, 1 - slot) if j + 1 < NCHUNK else None
        cur.wait()
        if out_copies[slot] is not None:
            out_copies[slot].wait()
        extract(j, slot)
        out_copies[slot] = pltpu.async_copy(
            row_bufs[slot],
            out_hbm.at[pl.ds(base + j * CHUNK, CHUNK)],
            out_sems[slot],
        )
    for c in out_copies:
        if c is not None:
            c.wait()


@jax.jit
def _gather(line_idx, parity, entity_table2):
    mesh = plsc.VectorSubcoreMesh(
        core_axis_name="c", subcore_axis_name="s",
        num_cores=NUM_CORES, num_subcores=NUM_SUBCORES,
    )
    return pl.kernel(
        _gather_body,
        out_type=jax.ShapeDtypeStruct((BATCH, DIM), jnp.float32),
        mesh=mesh,
        compiler_params=pltpu.CompilerParams(use_tc_tiling_on_sc=True),
        scratch_types=[
            pltpu.VMEM((NCHUNK, CHUNK), jnp.int32),
            pltpu.VMEM((B_PER_W,), jnp.int32),
            pltpu.VMEM((CHUNK, 2 * DIM), jnp.float32),
            pltpu.VMEM((CHUNK, 2 * DIM), jnp.float32),
            pltpu.VMEM((CHUNK, DIM), jnp.float32),
            pltpu.VMEM((CHUNK, DIM), jnp.float32),
            pltpu.SemaphoreType.DMA,
            pltpu.SemaphoreType.DMA,
            pltpu.SemaphoreType.DMA,
            pltpu.SemaphoreType.DMA,
        ],
    )(line_idx, parity, entity_table2)


def kernel(entities, entity_table):
    idx = entities.astype(jnp.int32)
    line_idx = (idx // 2).reshape(NW, NCHUNK, CHUNK)
    parity = (idx % 2).reshape(NW, B_PER_W)
    tab2 = jnp.concatenate([entity_table[0::2], entity_table[1::2]], axis=1)
    return _gather(line_idx, parity, tab2)


# speedup vs baseline: 11.1465x; 11.1465x over previous
import functools

import jax
import jax.numpy as jnp
from jax import lax
from jax.experimental import pallas as pl
from jax.experimental.pallas import tpu as pltpu
from jax.experimental.pallas import tpu_sc as plsc

BATCH = 16384
DIM = 64
NUM_CORES = 1
NUM_SUBCORES = 16
NW = NUM_CORES * NUM_SUBCORES
B_PER_W = BATCH // NW
CHUNK = 128
NCHUNK = B_PER_W // CHUNK
GRP = 16


def _gather_body(lidx_hbm, par_hbm, tab_hbm, out_hbm,
                 lidx_v, par_v, pairs_a, pairs_b, rows_a, rows_b,
                 sem_a, sem_b, osem_a, osem_b):
    wid = lax.axis_index("s") * NUM_CORES + lax.axis_index("c")
    base = wid * B_PER_W
    pltpu.sync_copy(lidx_hbm.at[wid], lidx_v)
    pltpu.sync_copy(par_hbm.at[wid], par_v)
    pair_bufs = (pairs_a, pairs_b)
    pair_sems = (sem_a, sem_b)
    row_bufs = (rows_a, rows_b)
    out_sems = (osem_a, osem_b)

    def fire(j, slot):
        return pltpu.async_copy(
            tab_hbm.at[lidx_v.at[j]], pair_bufs[slot], pair_sems[slot]
        )

    def extract(j, slot):
        buf = pair_bufs[slot]
        rows = row_bufs[slot]

        def grp_body(g, carry):
            voff = par_v[pl.ds(j * CHUNK + g * GRP, GRP)] * DIM
            for l in range(GRP):
                off = voff[l]
                for c in range(DIM // 16):
                    rows[g * GRP + l, pl.ds(c * 16, 16)] = (
                        buf[g * GRP + l, pl.ds(off + c * 16, 16)]
                    )
            return carry

        lax.fori_loop(0, CHUNK // GRP, grp_body, 0)

    out_copies = [None, None]
    pending = fire(0, 0)
    for j in range(NCHUNK):
        slot = j % 2
        cur = pending
        pending = fire(j + 1, 1 - slot) if j + 1 < NCHUNK else None
        cur.wait()
        if out_copies[slot] is not None:
            out_copies[slot].wait()
        extract(j, slot)
        out_copies[slot] = pltpu.async_copy(
            row_bufs[slot],
            out_hbm.at[pl.ds(base + j * CHUNK, CHUNK)],
            out_sems[slot],
        )
    for c in out_copies:
        if c is not None:
            c.wait()


@jax.jit
def _gather(line_idx, parity, entity_table2):
    mesh = plsc.VectorSubcoreMesh(
        core_axis_name="c", subcore_axis_name="s",
        num_cores=NUM_CORES, num_subcores=NUM_SUBCORES,
    )
    return pl.kernel(
        _gather_body,
        out_type=jax.ShapeDtypeStruct((BATCH, DIM), jnp.float32),
        mesh=mesh,
        compiler_params=pltpu.CompilerParams(use_tc_tiling_on_sc=True),
        scratch_types=[
            pltpu.VMEM((NCHUNK, CHUNK), jnp.int32),
            pltpu.VMEM((B_PER_W,), jnp.int32),
            pltpu.VMEM((CHUNK, 2 * DIM), jnp.float32),
            pltpu.VMEM((CHUNK, 2 * DIM), jnp.float32),
            pltpu.VMEM((CHUNK, DIM), jnp.float32),
            pltpu.VMEM((CHUNK, DIM), jnp.float32),
            pltpu.SemaphoreType.DMA,
            pltpu.SemaphoreType.DMA,
            pltpu.SemaphoreType.DMA,
            pltpu.SemaphoreType.DMA,
        ],
    )(line_idx, parity, entity_table2)


def kernel(entities, entity_table):
    idx = entities.astype(jnp.int32)
    line_idx = (idx // 2).reshape(NW, NCHUNK, CHUNK)
    parity = (idx % 2).reshape(NW, B_PER_W)
    n2 = entity_table.shape[0] // 2
    tab2 = (entity_table.T.reshape(DIM, n2, 2)
            .transpose(1, 2, 0).reshape(n2, 2 * DIM))
    return _gather(line_idx, parity, tab2)


# speedup vs baseline: 35.9695x; 3.2270x over previous
import functools

import jax
import jax.numpy as jnp
from jax import lax
from jax.experimental import pallas as pl
from jax.experimental.pallas import tpu as pltpu
from jax.experimental.pallas import tpu_sc as plsc

BATCH = 16384
DIM = 64
NUM_CORES = 2
NUM_SUBCORES = 16
NW = NUM_CORES * NUM_SUBCORES
B_PER_W = BATCH // NW


def _gather_body(idx_hbm, tab_hbm, out_hbm, idx_v, rows_v, sem):
    wid = lax.axis_index("s") * NUM_CORES + lax.axis_index("c")
    base = wid * B_PER_W
    pltpu.sync_copy(idx_hbm.at[wid], idx_v)

    def body(g, carry):
        vidx = idx_v[pl.ds(g * 16, 16)]
        vt = jnp.right_shift(vidx, 3)
        vr = jnp.bitwise_and(vidx, 7)
        for l in range(16):
            pltpu.async_copy(
                tab_hbm.at[vt[l], vr[l]], rows_v.at[g * 16 + l], sem
            )
        return carry

    lax.fori_loop(0, B_PER_W // 16, body, 0)
    pltpu.make_async_copy(out_hbm.at[pl.ds(0, B_PER_W)], rows_v, sem).wait()
    pltpu.sync_copy(rows_v, out_hbm.at[pl.ds(base, B_PER_W)])


@jax.jit
def _gather(entities_blocks, entity_table3):
    mesh = plsc.VectorSubcoreMesh(
        core_axis_name="c", subcore_axis_name="s",
        num_cores=NUM_CORES, num_subcores=NUM_SUBCORES,
    )
    return pl.kernel(
        _gather_body,
        out_type=jax.ShapeDtypeStruct((BATCH, DIM), jnp.float32),
        mesh=mesh,
        compiler_params=pltpu.CompilerParams(use_tc_tiling_on_sc=True),
        scratch_types=[
            pltpu.VMEM((B_PER_W,), jnp.int32),
            pltpu.VMEM((B_PER_W, DIM), jnp.float32),
            pltpu.SemaphoreType.DMA,
        ],
    )(entities_blocks, entity_table3)


def kernel(entities, entity_table):
    idx = entities.astype(jnp.int32).reshape(NW, B_PER_W)
    tab3 = entity_table.reshape(entity_table.shape[0] // 8, 8, DIM)
    return _gather(idx, tab3)
